# baseline (speedup 1.0000x reference)
import jax
import jax.numpy as jnp
from jax import lax
from jax.experimental import pallas as pl
from jax.experimental.pallas import tpu as pltpu
from jax.experimental.pallas import tpu_sc as plsc

_N = 10000
_E = 320000
_G = 16

_NTILES = 16
_NCORES = 2
_NPAD = 10240
_EPW = _E // (_NCORES * _NTILES)
_CH = 80
_NCHUNK = _EPW // _CH
_NP = 124
_NBUF = 4
_RPT = _NPAD // _NTILES
_RCH = 64
_NRC = _RPT // _RCH


def _fc1_body(x_ref, w_ref, b_ref, o_ref):
    o_ref[...] = jnp.maximum(
        jnp.dot(x_ref[...], w_ref[...], preferred_element_type=jnp.float32)
        + b_ref[...], 0.0)


def _fc1(xp, W1p, b1r):
    nb = 2048
    return pl.pallas_call(
        _fc1_body,
        grid=(_NPAD // nb,),
        in_specs=[
            pl.BlockSpec((nb, 128), lambda i: (i, 0)),
            pl.BlockSpec((128, 128), lambda i: (0, 0)),
            pl.BlockSpec((1, 128), lambda i: (0, 0)),
        ],
        out_specs=pl.BlockSpec((nb, 128), lambda i: (i, 0)),
        out_shape=jax.ShapeDtypeStruct((_NPAD, 128), jnp.float32),
    )(xp, W1p, b1r)


def _segsum_body(h_hbm, src_hbm, dst_hbm, z_hbm, out_hbm,
                 acc, sv0, sv1, sv2, sv3, dv0, dv1, dv2, dv3,
                 rv0, rv1, obuf, isem, gsem, ssem):
    c = lax.axis_index("c")
    s = lax.axis_index("s")
    w = c * _NTILES + s
    rbase = s * _RPT
    src_v = [sv0, sv1, sv2, sv3]
    dst_v = [dv0, dv1, dv2, dv3]
    rows_v = [rv0, rv1]

    pltpu.sync_copy(z_hbm, obuf)

    def zk(k, carry):
        pltpu.sync_copy(obuf, acc.at[pl.ds(rbase + k * _RCH, _RCH)])
        return carry
    lax.fori_loop(0, _NRC, zk, 0)
    plsc.subcore_barrier()

    ebase = w * _EPW

    def issue_idx(j, bi):
        base = ebase + j * _CH
        pltpu.async_copy(src_hbm.at[pl.ds(base, _CH)], src_v[bi],
                         isem.at[bi])
        pltpu.async_copy(dst_hbm.at[pl.ds(base, _CH)], dst_v[bi],
                         isem.at[bi])

    def wait_idx(j, bi):
        base = ebase + j * _CH
        pltpu.make_async_copy(src_hbm.at[pl.ds(base, _CH)], src_v[bi],
                              isem.at[bi]).wait()
        pltpu.make_async_copy(dst_hbm.at[pl.ds(base, _CH)], dst_v[bi],
                              isem.at[bi]).wait()

    def issue_gather(bi, br):
        pltpu.async_copy(h_hbm.at[src_v[bi]], rows_v[br], gsem.at[br])

    def wait_gather(bi, br):
        pltpu.make_async_copy(h_hbm.at[src_v[bi]], rows_v[br],
                              gsem.at[br]).wait()

    def issue_scatter(bi, br):
        pltpu.async_copy(rows_v[br], acc.at[dst_v[bi]], ssem.at[br],
                         add=True)

    def wait_scatter(bi, br):
        pltpu.make_async_copy(rows_v[br], acc.at[dst_v[bi]],
                              ssem.at[br]).wait()

    issue_idx(0, 0)
    issue_idx(1, 1)
    issue_idx(2, 2)
    wait_idx(0, 0)
    issue_gather(0, 0)
    wait_gather(0, 0)
    issue_scatter(0, 0)
    issue_idx(3, 3)
    wait_idx(1, 1)
    issue_gather(1, 1)
    wait_gather(1, 1)
    issue_scatter(1, 1)
    wait_scatter(0, 0)
    issue_idx(4, 0)
    wait_idx(2, 2)
    issue_gather(2, 0)
    wait_gather(2, 0)
    issue_scatter(2, 0)
    wait_scatter(1, 1)
    issue_idx(5, 1)
    wait_idx(3, 3)
    issue_gather(3, 1)
    wait_gather(3, 1)
    issue_scatter(3, 1)
    wait_scatter(2, 0)
    issue_idx(6, 2)
    wait_idx(4, 0)
    issue_gather(0, 0)

    def group(jo, carry):
        for b in range(4):
            j = jo * 4 + b
            br = b % 2
            wait_gather(b, br)
            issue_scatter(b, br)
            wait_scatter((b - 1) % 4, br ^ 1)
            issue_idx(j + 3, (b - 1) % 4)
            wait_idx(j + 1, (b + 1) % 4)
            issue_gather((b + 1) % 4, br ^ 1)
        return carry
    lax.fori_loop(1, (_NP - 4) // 4, group, 0)

    wait_gather(0, 0)
    issue_scatter(0, 0)
    wait_scatter(3, 1)
    issue_idx(_NP - 1, 3)
    wait_idx(_NP - 3, 1)
    issue_gather(1, 1)
    wait_gather(1, 1)
    issue_scatter(1, 1)
    wait_scatter(0, 0)
    wait_idx(_NP - 2, 2)
    issue_gather(2, 0)
    wait_gather(2, 0)
    issue_scatter(2, 0)
    wait_scatter(1, 1)
    wait_idx(_NP - 1, 3)
    issue_gather(3, 1)
    wait_gather(3, 1)
    issue_scatter(3, 1)
    wait_scatter(2, 0)
    wait_scatter(3, 1)

    fb = ebase + _NP * _CH
    pltpu.sync_copy(src_hbm.at[pl.ds(fb, _CH)], sv0)
    pltpu.sync_copy(dst_hbm.at[pl.ds(fb, _CH)], dv0)
    pltpu.async_copy(h_hbm.at[sv0], rv0, gsem.at[0]).wait()
    pltpu.sync_copy(rv0, acc.at[dv0], add=True)
    plsc.subcore_barrier()

    def ok(k, carry):
        r = rbase + k * _RCH
        pltpu.sync_copy(acc.at[pl.ds(r, _RCH)], obuf)
        pltpu.sync_copy(obuf, out_hbm.at[c, pl.ds(r, _RCH)])
        return carry
    lax.fori_loop(0, _NRC, ok, 0)


def _segsum(h, src, dst, zrows):
    mesh = plsc.VectorSubcoreMesh(core_axis_name="c", subcore_axis_name="s")
    ivec = pltpu.VMEM((_CH,), jnp.int32)
    rbuf = pltpu.VMEM((_CH, 128), jnp.float32)
    f = pl.kernel(
        _segsum_body,
        mesh=mesh,
        out_type=jax.ShapeDtypeStruct((_NCORES, _NPAD, 128), jnp.float32),
        scratch_types=[
            pltpu.VMEM_SHARED((_NPAD, 128), jnp.float32),
            ivec, ivec, ivec, ivec,
            ivec, ivec, ivec, ivec,
            rbuf, rbuf,
            pltpu.VMEM((_RCH, 128), jnp.float32),
            pltpu.SemaphoreType.DMA((4,)),
            pltpu.SemaphoreType.DMA((2,)),
            pltpu.SemaphoreType.DMA((2,)),
        ],
    )
    return f(h, src, dst, zrows)


def _tail_body(h_ref, s0_ref, s1_ref, bb_ref, wr_ref, wn_ref, bc_ref,
               w2_ref, b2_ref, wmu_ref, bmu_ref, wlv_ref, blv_ref,
               mu_ref, lv_ref, acc_ref):
    i = pl.program_id(0)
    nsteps = pl.num_programs(0)

    @pl.when(i == 0)
    def _():
        acc_ref[...] = jnp.full((_G, 128), -jnp.inf, jnp.float32)

    h = h_ref[...]
    sagg = s0_ref[0] + s1_ref[0]
    h2 = (jnp.dot(h, wr_ref[...], preferred_element_type=jnp.float32)
          + jnp.dot(sagg, wn_ref[...], preferred_element_type=jnp.float32)
          + bc_ref[...])
    h3 = jnp.maximum(
        jnp.dot(h2, w2_ref[...], preferred_element_type=jnp.float32)
        + b2_ref[...], 0.0)

    bb = bb_ref[...][:, 0:1]
    for g in range(_G):
        red = jnp.max(jnp.where(bb == g, h3, -jnp.inf), axis=0,
                      keepdims=True)
        acc_ref[pl.ds(g, 1), :] = jnp.maximum(acc_ref[pl.ds(g, 1), :], red)

    @pl.when(i == nsteps - 1)
    def _():
        pooled = acc_ref[...]
        mu_ref[...] = (jnp.dot(pooled, wmu_ref[...],
                               preferred_element_type=jnp.float32)
                       + bmu_ref[...])
        lv_ref[...] = (jnp.dot(pooled, wlv_ref[...],
                               preferred_element_type=jnp.float32)
                       + blv_ref[...])


def _tail(h, partials, bb, Wroot, Wnbr, bcr, W2, b2r, Wmu, bmur, Wlv, blvr):
    nb = 2000
    full = lambda r, c: pl.BlockSpec((r, c), lambda i: (0, 0))
    return pl.pallas_call(
        _tail_body,
        grid=(_N // nb,),
        in_specs=[
            pl.BlockSpec((nb, 128), lambda i: (i, 0)),
            pl.BlockSpec((1, nb, 128), lambda i: (0, i, 0)),
            pl.BlockSpec((1, nb, 128), lambda i: (1, i, 0)),
            pl.BlockSpec((nb, 8), lambda i: (i, 0)),
            full(128, 256),
            full(128, 256),
            full(1, 256),
            full(256, 128),
            full(1, 128),
            full(128, 128),
            full(1, 128),
            full(128, 128),
            full(1, 128),
        ],
        out_specs=[full(_G, 128), full(_G, 128)],
        out_shape=[jax.ShapeDtypeStruct((_G, 128), jnp.float32),
                   jax.ShapeDtypeStruct((_G, 128), jnp.float32)],
        scratch_shapes=[pltpu.VMEM((_G, 128), jnp.float32)],
    )(h, partials, partials, bb, Wroot, Wnbr, bcr, W2, b2r, Wmu, bmur,
      Wlv, blvr)


def kernel(x, edge_index, batch, W1, b1, Wroot, Wnbr, bconv, W2, b2,
           Wmu, bmu, Wlv, blv):
    src = edge_index[0]
    dst = edge_index[1]
    xp = jnp.pad(x, ((0, _NPAD - _N), (0, 125)))
    W1p = jnp.pad(W1, ((0, 125), (0, 0)))
    h = _fc1(xp, W1p, b1.reshape(1, 128))
    zrows = jnp.zeros((_RCH, 128), jnp.float32)
    partials = _segsum(h, src, dst, zrows)
    bb = jnp.broadcast_to(batch[:, None], (_N, 8))
    mu, lv = _tail(h, partials, bb, Wroot, Wnbr,
                   bconv.reshape(1, 256), W2, b2.reshape(1, 128),
                   Wmu, bmu.reshape(1, 128), Wlv, blv.reshape(1, 128))
    return (mu, lv)


# speedup vs baseline: 1.1179x; 1.1179x over previous
import jax
import jax.numpy as jnp
from jax import lax
from jax.experimental import pallas as pl
from jax.experimental.pallas import tpu as pltpu
from jax.experimental.pallas import tpu_sc as plsc

_N = 10000
_E = 320000
_G = 16

_NTILES = 16
_NCORES = 2
_NPAD = 10240
_EPAD = 327680
_EPW = _EPAD // (_NCORES * _NTILES)
_CH = 128
_NCHUNK = _EPW // _CH
_NP = _NCHUNK
_NBUF = 4
_RPT = _NPAD // _NTILES
_RCH = 64
_NRC = _RPT // _RCH


def _fc1_body(x_ref, w_ref, b_ref, o_ref):
    o_ref[...] = jnp.maximum(
        jnp.dot(x_ref[...], w_ref[...], preferred_element_type=jnp.float32)
        + b_ref[...], 0.0)


def _fc1(xp, W1p, b1r):
    nb = 2048
    return pl.pallas_call(
        _fc1_body,
        grid=(_NPAD // nb,),
        in_specs=[
            pl.BlockSpec((nb, 128), lambda i: (i, 0)),
            pl.BlockSpec((128, 128), lambda i: (0, 0)),
            pl.BlockSpec((1, 128), lambda i: (0, 0)),
        ],
        out_specs=pl.BlockSpec((nb, 128), lambda i: (i, 0)),
        out_shape=jax.ShapeDtypeStruct((_NPAD, 128), jnp.float32),
    )(xp, W1p, b1r)


def _segsum_body(h_hbm, src_hbm, dst_hbm, z_hbm, out_hbm,
                 acc, sv0, sv1, sv2, sv3, dv0, dv1, dv2, dv3,
                 rv0, rv1, obuf, isem, gsem, ssem):
    c = lax.axis_index("c")
    s = lax.axis_index("s")
    w = c * _NTILES + s
    rbase = s * _RPT
    src_v = [sv0, sv1, sv2, sv3]
    dst_v = [dv0, dv1, dv2, dv3]
    rows_v = [rv0, rv1]

    pltpu.sync_copy(z_hbm, obuf)

    def zk(k, carry):
        pltpu.sync_copy(obuf, acc.at[pl.ds(rbase + k * _RCH, _RCH)])
        return carry
    lax.fori_loop(0, _NRC, zk, 0)
    plsc.subcore_barrier()

    ebase = w * _EPW

    def issue_idx(j, bi):
        base = ebase + j * _CH
        pltpu.async_copy(src_hbm.at[pl.ds(base, _CH)], src_v[bi],
                         isem.at[bi])
        pltpu.async_copy(dst_hbm.at[pl.ds(base, _CH)], dst_v[bi],
                         isem.at[bi])

    def wait_idx(j, bi):
        base = ebase + j * _CH
        pltpu.make_async_copy(src_hbm.at[pl.ds(base, _CH)], src_v[bi],
                              isem.at[bi]).wait()
        pltpu.make_async_copy(dst_hbm.at[pl.ds(base, _CH)], dst_v[bi],
                              isem.at[bi]).wait()

    def issue_gather(bi, br):
        pltpu.async_copy(h_hbm.at[src_v[bi]], rows_v[br], gsem.at[br])

    def wait_gather(bi, br):
        pltpu.make_async_copy(h_hbm.at[src_v[bi]], rows_v[br],
                              gsem.at[br]).wait()

    def issue_scatter(bi, br):
        pltpu.async_copy(rows_v[br], acc.at[dst_v[bi]], ssem.at[br],
                         add=True)

    def wait_scatter(bi, br):
        pltpu.make_async_copy(rows_v[br], acc.at[dst_v[bi]],
                              ssem.at[br]).wait()

    issue_idx(0, 0)
    issue_idx(1, 1)
    issue_idx(2, 2)
    wait_idx(0, 0)
    issue_gather(0, 0)
    wait_gather(0, 0)
    issue_scatter(0, 0)
    issue_idx(3, 3)
    wait_idx(1, 1)
    issue_gather(1, 1)
    wait_gather(1, 1)
    issue_scatter(1, 1)
    wait_scatter(0, 0)
    issue_idx(4, 0)
    wait_idx(2, 2)
    issue_gather(2, 0)
    wait_gather(2, 0)
    issue_scatter(2, 0)
    wait_scatter(1, 1)
    issue_idx(5, 1)
    wait_idx(3, 3)
    issue_gather(3, 1)
    wait_gather(3, 1)
    issue_scatter(3, 1)
    wait_scatter(2, 0)
    issue_idx(6, 2)
    wait_idx(4, 0)
    issue_gather(0, 0)

    def group(jo, carry):
        for b in range(4):
            j = jo * 4 + b
            br = b % 2
            wait_gather(b, br)
            issue_scatter(b, br)
            wait_scatter((b - 1) % 4, br ^ 1)
            issue_idx(j + 3, (b - 1) % 4)
            wait_idx(j + 1, (b + 1) % 4)
            issue_gather((b + 1) % 4, br ^ 1)
        return carry
    lax.fori_loop(1, (_NP - 4) // 4, group, 0)

    wait_gather(0, 0)
    issue_scatter(0, 0)
    wait_scatter(3, 1)
    issue_idx(_NP - 1, 3)
    wait_idx(_NP - 3, 1)
    issue_gather(1, 1)
    wait_gather(1, 1)
    issue_scatter(1, 1)
    wait_scatter(0, 0)
    wait_idx(_NP - 2, 2)
    issue_gather(2, 0)
    wait_gather(2, 0)
    issue_scatter(2, 0)
    wait_scatter(1, 1)
    wait_idx(_NP - 1, 3)
    issue_gather(3, 1)
    wait_gather(3, 1)
    issue_scatter(3, 1)
    wait_scatter(2, 0)
    wait_scatter(3, 1)
    plsc.subcore_barrier()

    def ok(k, carry):
        r = rbase + k * _RCH
        pltpu.sync_copy(acc.at[pl.ds(r, _RCH)], obuf)
        pltpu.sync_copy(obuf, out_hbm.at[c, pl.ds(r, _RCH)])
        return carry
    lax.fori_loop(0, _NRC, ok, 0)


def _segsum(h, src, dst, zrows):
    mesh = plsc.VectorSubcoreMesh(core_axis_name="c", subcore_axis_name="s")
    ivec = pltpu.VMEM((_CH,), jnp.int32)
    rbuf = pltpu.VMEM((_CH, 128), jnp.float32)
    f = pl.kernel(
        _segsum_body,
        mesh=mesh,
        out_type=jax.ShapeDtypeStruct((_NCORES, _NPAD, 128), jnp.float32),
        scratch_types=[
            pltpu.VMEM_SHARED((_NPAD, 128), jnp.float32),
            ivec, ivec, ivec, ivec,
            ivec, ivec, ivec, ivec,
            rbuf, rbuf,
            pltpu.VMEM((_RCH, 128), jnp.float32),
            pltpu.SemaphoreType.DMA((4,)),
            pltpu.SemaphoreType.DMA((2,)),
            pltpu.SemaphoreType.DMA((2,)),
        ],
    )
    return f(h, src, dst, zrows)


def _tail_body(h_ref, s0_ref, s1_ref, bb_ref, wr_ref, wn_ref, bc_ref,
               w2_ref, b2_ref, wmu_ref, bmu_ref, wlv_ref, blv_ref,
               mu_ref, lv_ref, acc_ref):
    i = pl.program_id(0)
    nsteps = pl.num_programs(0)

    @pl.when(i == 0)
    def _():
        acc_ref[...] = jnp.full((_G, 128), -jnp.inf, jnp.float32)

    h = h_ref[...]
    sagg = s0_ref[0] + s1_ref[0]
    h2 = (jnp.dot(h, wr_ref[...], preferred_element_type=jnp.float32)
          + jnp.dot(sagg, wn_ref[...], preferred_element_type=jnp.float32)
          + bc_ref[...])
    h3 = jnp.maximum(
        jnp.dot(h2, w2_ref[...], preferred_element_type=jnp.float32)
        + b2_ref[...], 0.0)

    bb = bb_ref[...][:, 0:1]
    for g in range(_G):
        red = jnp.max(jnp.where(bb == g, h3, -jnp.inf), axis=0,
                      keepdims=True)
        acc_ref[pl.ds(g, 1), :] = jnp.maximum(acc_ref[pl.ds(g, 1), :], red)

    @pl.when(i == nsteps - 1)
    def _():
        pooled = acc_ref[...]
        mu_ref[...] = (jnp.dot(pooled, wmu_ref[...],
                               preferred_element_type=jnp.float32)
                       + bmu_ref[...])
        lv_ref[...] = (jnp.dot(pooled, wlv_ref[...],
                               preferred_element_type=jnp.float32)
                       + blv_ref[...])


def _tail(h, partials, bb, Wroot, Wnbr, bcr, W2, b2r, Wmu, bmur, Wlv, blvr):
    nb = 2000
    full = lambda r, c: pl.BlockSpec((r, c), lambda i: (0, 0))
    return pl.pallas_call(
        _tail_body,
        grid=(_N // nb,),
        in_specs=[
            pl.BlockSpec((nb, 128), lambda i: (i, 0)),
            pl.BlockSpec((1, nb, 128), lambda i: (0, i, 0)),
            pl.BlockSpec((1, nb, 128), lambda i: (1, i, 0)),
            pl.BlockSpec((nb, 8), lambda i: (i, 0)),
            full(128, 256),
            full(128, 256),
            full(1, 256),
            full(256, 128),
            full(1, 128),
            full(128, 128),
            full(1, 128),
            full(128, 128),
            full(1, 128),
        ],
        out_specs=[full(_G, 128), full(_G, 128)],
        out_shape=[jax.ShapeDtypeStruct((_G, 128), jnp.float32),
                   jax.ShapeDtypeStruct((_G, 128), jnp.float32)],
        scratch_shapes=[pltpu.VMEM((_G, 128), jnp.float32)],
    )(h, partials, partials, bb, Wroot, Wnbr, bcr, W2, b2r, Wmu, bmur,
      Wlv, blvr)


def kernel(x, edge_index, batch, W1, b1, Wroot, Wnbr, bconv, W2, b2,
           Wmu, bmu, Wlv, blv):
    npad_e = _EPAD - _E
    pad_ids = _N + (jnp.arange(npad_e, dtype=jnp.int32) & 127)
    src = jnp.concatenate([edge_index[0], pad_ids])
    dst = jnp.concatenate([edge_index[1], pad_ids])
    xp = jnp.pad(x, ((0, _NPAD - _N), (0, 125)))
    W1p = jnp.pad(W1, ((0, 125), (0, 0)))
    h = _fc1(xp, W1p, b1.reshape(1, 128))
    zrows = jnp.zeros((_RCH, 128), jnp.float32)
    partials = _segsum(h, src, dst, zrows)
    bb = jnp.broadcast_to(batch[:, None], (_N, 8))
    mu, lv = _tail(h, partials, bb, Wroot, Wnbr,
                   bconv.reshape(1, 256), W2, b2.reshape(1, 128),
                   Wmu, bmu.reshape(1, 128), Wlv, blv.reshape(1, 128))
    return (mu, lv)


# speedup vs baseline: 1.1807x; 1.0562x over previous
import jax
import jax.numpy as jnp
from jax import lax
from jax.experimental import pallas as pl
from jax.experimental.pallas import tpu as pltpu
from jax.experimental.pallas import tpu_sc as plsc

_N = 10000
_E = 320000
_G = 16

_NTILES = 16
_NCORES = 2
_NPAD = 10240
_EPAD = 327680
_EPW = _EPAD // (_NCORES * _NTILES)
_CH = 128
_NP = _EPW // _CH
_RPT = _NPAD // _NTILES
_RCH = 64
_NRC = _RPT // _RCH


def _fc1_body(x_ref, w_ref, b_ref, o_ref):
    o_ref[...] = jnp.maximum(
        jnp.dot(x_ref[...], w_ref[...], preferred_element_type=jnp.float32)
        + b_ref[...], 0.0)


def _fc1(xp, W1p, b1r):
    nb = 2048
    return pl.pallas_call(
        _fc1_body,
        grid=(_NPAD // nb,),
        in_specs=[
            pl.BlockSpec((nb, 8), lambda i: (i, 0)),
            pl.BlockSpec((8, 128), lambda i: (0, 0)),
            pl.BlockSpec((1, 128), lambda i: (0, 0)),
        ],
        out_specs=pl.BlockSpec((nb, 128), lambda i: (i, 0)),
        out_shape=jax.ShapeDtypeStruct((_NPAD, 128), jnp.float32),
    )(xp, W1p, b1r)


def _segsum_body(h_hbm, src_hbm, dst_hbm, z_hbm, out_hbm,
                 acc, sv0, sv1, sv2, sv3, dv0, dv1, dv2, dv3,
                 rv0, rv1, obuf, isem, gsem, ssem):
    c = lax.axis_index("c")
    s = lax.axis_index("s")
    w = c * _NTILES + s
    rbase = s * _RPT
    src_v = [sv0, sv1, sv2, sv3]
    dst_v = [dv0, dv1, dv2, dv3]
    rows_v = [rv0, rv1]

    pltpu.sync_copy(z_hbm, obuf)

    def zk(k, carry):
        pltpu.sync_copy(obuf, acc.at[pl.ds(rbase + k * _RCH, _RCH)])
        return carry
    lax.fori_loop(0, _NRC, zk, 0)
    plsc.subcore_barrier()

    ebase = w * _EPW

    def issue_idx(j, bi):
        base = ebase + j * _CH
        pltpu.async_copy(src_hbm.at[pl.ds(base, _CH)], src_v[bi],
                         isem.at[bi])
        pltpu.async_copy(dst_hbm.at[pl.ds(base, _CH)], dst_v[bi],
                         isem.at[bi])

    def wait_idx(j, bi):
        base = ebase + j * _CH
        pltpu.make_async_copy(src_hbm.at[pl.ds(base, _CH)], src_v[bi],
                              isem.at[bi]).wait()
        pltpu.make_async_copy(dst_hbm.at[pl.ds(base, _CH)], dst_v[bi],
                              isem.at[bi]).wait()

    def issue_gather(bi, br):
        pltpu.async_copy(h_hbm.at[src_v[bi]], rows_v[br], gsem.at[br])

    def wait_gather(bi, br):
        pltpu.make_async_copy(h_hbm.at[src_v[bi]], rows_v[br],
                              gsem.at[br]).wait()

    def issue_scatter(bi, br):
        pltpu.async_copy(rows_v[br], acc.at[dst_v[bi]], ssem.at[br],
                         add=True)

    def wait_scatter(bi, br):
        pltpu.make_async_copy(rows_v[br], acc.at[dst_v[bi]],
                              ssem.at[br]).wait()

    issue_idx(0, 0)
    issue_idx(1, 1)
    issue_idx(2, 2)
    wait_idx(0, 0)
    issue_gather(0, 0)
    wait_gather(0, 0)
    issue_scatter(0, 0)
    issue_idx(3, 3)
    wait_idx(1, 1)
    issue_gather(1, 1)
    wait_gather(1, 1)
    issue_scatter(1, 1)
    wait_scatter(0, 0)
    issue_idx(4, 0)
    wait_idx(2, 2)
    issue_gather(2, 0)
    wait_gather(2, 0)
    issue_scatter(2, 0)
    wait_scatter(1, 1)
    issue_idx(5, 1)
    wait_idx(3, 3)
    issue_gather(3, 1)
    wait_gather(3, 1)
    issue_scatter(3, 1)
    wait_scatter(2, 0)
    issue_idx(6, 2)
    wait_idx(4, 0)
    issue_gather(0, 0)

    def group(jo, carry):
        for b in range(4):
            j = jo * 4 + b
            br = b % 2
            wait_gather(b, br)
            issue_scatter(b, br)
            wait_scatter((b - 1) % 4, br ^ 1)
            issue_idx(j + 3, (b - 1) % 4)
            wait_idx(j + 1, (b + 1) % 4)
            issue_gather((b + 1) % 4, br ^ 1)
        return carry
    lax.fori_loop(1, (_NP - 4) // 4, group, 0)

    wait_gather(0, 0)
    issue_scatter(0, 0)
    wait_scatter(3, 1)
    issue_idx(_NP - 1, 3)
    wait_idx(_NP - 3, 1)
    issue_gather(1, 1)
    wait_gather(1, 1)
    issue_scatter(1, 1)
    wait_scatter(0, 0)
    wait_idx(_NP - 2, 2)
    issue_gather(2, 0)
    wait_gather(2, 0)
    issue_scatter(2, 0)
    wait_scatter(1, 1)
    wait_idx(_NP - 1, 3)
    issue_gather(3, 1)
    wait_gather(3, 1)
    issue_scatter(3, 1)
    wait_scatter(2, 0)
    wait_scatter(3, 1)
    plsc.subcore_barrier()

    def ok(k, carry):
        r = rbase + k * _RCH
        pltpu.sync_copy(acc.at[pl.ds(r, _RCH)], obuf)
        pltpu.sync_copy(obuf, out_hbm.at[c, pl.ds(r, _RCH)])
        return carry
    lax.fori_loop(0, _NRC, ok, 0)


def _segsum(h, src, dst, zrows):
    mesh = plsc.VectorSubcoreMesh(core_axis_name="c", subcore_axis_name="s")
    ivec = pltpu.VMEM((_CH,), jnp.int32)
    rbuf = pltpu.VMEM((_CH, 128), jnp.float32)
    f = pl.kernel(
        _segsum_body,
        mesh=mesh,
        out_type=jax.ShapeDtypeStruct((_NCORES, _NPAD, 128), jnp.float32),
        scratch_types=[
            pltpu.VMEM_SHARED((_NPAD, 128), jnp.float32),
            ivec, ivec, ivec, ivec,
            ivec, ivec, ivec, ivec,
            rbuf, rbuf,
            pltpu.VMEM((_RCH, 128), jnp.float32),
            pltpu.SemaphoreType.DMA((4,)),
            pltpu.SemaphoreType.DMA((2,)),
            pltpu.SemaphoreType.DMA((2,)),
        ],
    )
    return f(h, src, dst, zrows)


def _tail_body(h_ref, s0_ref, s1_ref, bb_ref, wr_ref, wn_ref, bc_ref,
               w2_ref, b2_ref, wmu_ref, bmu_ref, wlv_ref, blv_ref,
               mu_ref, lv_ref, acc_ref):
    i = pl.program_id(0)
    nsteps = pl.num_programs(0)

    @pl.when(i == 0)
    def _():
        acc_ref[...] = jnp.full((_G, 128), -jnp.inf, jnp.float32)

    h = h_ref[...]
    sagg = s0_ref[0] + s1_ref[0]
    h2 = (jnp.dot(h, wr_ref[...], preferred_element_type=jnp.float32)
          + jnp.dot(sagg, wn_ref[...], preferred_element_type=jnp.float32)
          + bc_ref[...])
    h3 = jnp.maximum(
        jnp.dot(h2, w2_ref[...], preferred_element_type=jnp.float32)
        + b2_ref[...], 0.0)

    bb = bb_ref[...]
    glo = bb[0, 0]
    ghi = bb[bb.shape[0] - 1, 0]

    def gstep(g, carry):
        red = jnp.max(jnp.where(bb == g, h3, -jnp.inf), axis=0,
                      keepdims=True)
        cur = acc_ref[pl.ds(g, 1), :]
        acc_ref[pl.ds(g, 1), :] = jnp.maximum(cur, red)
        return carry
    lax.fori_loop(glo, ghi + 1, gstep, 0)

    @pl.when(i == nsteps - 1)
    def _():
        pooled = acc_ref[...]
        mu_ref[...] = (jnp.dot(pooled, wmu_ref[...],
                               preferred_element_type=jnp.float32)
                       + bmu_ref[...])
        lv_ref[...] = (jnp.dot(pooled, wlv_ref[...],
                               preferred_element_type=jnp.float32)
                       + blv_ref[...])


def _tail(h, partials, bb, Wroot, Wnbr, bcr, W2, b2r, Wmu, bmur, Wlv, blvr):
    nb = 2000
    full = lambda r, c: pl.BlockSpec((r, c), lambda i: (0, 0))
    return pl.pallas_call(
        _tail_body,
        grid=(_N // nb,),
        in_specs=[
            pl.BlockSpec((nb, 128), lambda i: (i, 0)),
            pl.BlockSpec((1, nb, 128), lambda i: (0, i, 0)),
            pl.BlockSpec((1, nb, 128), lambda i: (1, i, 0)),
            pl.BlockSpec((nb, 128), lambda i: (i, 0)),
            full(128, 256),
            full(128, 256),
            full(1, 256),
            full(256, 128),
            full(1, 128),
            full(128, 128),
            full(1, 128),
            full(128, 128),
            full(1, 128),
        ],
        out_specs=[full(_G, 128), full(_G, 128)],
        out_shape=[jax.ShapeDtypeStruct((_G, 128), jnp.float32),
                   jax.ShapeDtypeStruct((_G, 128), jnp.float32)],
        scratch_shapes=[pltpu.VMEM((_G, 128), jnp.float32)],
    )(h, partials, partials, bb, Wroot, Wnbr, bcr, W2, b2r, Wmu, bmur,
      Wlv, blvr)


def kernel(x, edge_index, batch, W1, b1, Wroot, Wnbr, bconv, W2, b2,
           Wmu, bmu, Wlv, blv):
    npad_e = _EPAD - _E
    pad2 = jnp.broadcast_to(
        (_N + jnp.arange(128, dtype=jnp.int32))[None, :],
        (npad_e // 128, 128))
    src = jnp.concatenate([edge_index[0].reshape(-1, 128), pad2]).reshape(-1)
    dst = jnp.concatenate([edge_index[1].reshape(-1, 128), pad2]).reshape(-1)
    xp = jnp.pad(x, ((0, _NPAD - _N), (0, 5)))
    W1p = jnp.pad(W1, ((0, 5), (0, 0)))
    h = _fc1(xp, W1p, b1.reshape(1, 128))
    zrows = jnp.zeros((_RCH, 128), jnp.float32)
    partials = _segsum(h, src, dst, zrows)
    bb = jnp.broadcast_to(batch[:, None], (_N, 128))
    mu, lv = _tail(h, partials, bb, Wroot, Wnbr,
                   bconv.reshape(1, 256), W2, b2.reshape(1, 128),
                   Wmu, bmu.reshape(1, 128), Wlv, blv.reshape(1, 128))
    return (mu, lv)


# speedup vs baseline: 1.1858x; 1.0043x over previous
import jax
import jax.numpy as jnp
from jax import lax
from jax.experimental import pallas as pl
from jax.experimental.pallas import tpu as pltpu
from jax.experimental.pallas import tpu_sc as plsc

_N = 10000
_E = 320000
_G = 16

_NTILES = 16
_NCORES = 2
_NPAD = 10240
_EPW = _E // (_NCORES * _NTILES)
_CH = 128
_NP = 76
_CT = 16
_RPT = _NPAD // _NTILES
_RCH = 64
_NRC = _RPT // _RCH


def _fc1_body(x_ref, w_ref, b_ref, o_ref):
    o_ref[...] = jnp.maximum(
        jnp.dot(x_ref[...], w_ref[...], preferred_element_type=jnp.float32)
        + b_ref[...], 0.0)


def _fc1(xp, W1p, b1r):
    nb = 2048
    return pl.pallas_call(
        _fc1_body,
        grid=(_NPAD // nb,),
        in_specs=[
            pl.BlockSpec((nb, 8), lambda i: (i, 0)),
            pl.BlockSpec((8, 128), lambda i: (0, 0)),
            pl.BlockSpec((1, 128), lambda i: (0, 0)),
        ],
        out_specs=pl.BlockSpec((nb, 128), lambda i: (i, 0)),
        out_shape=jax.ShapeDtypeStruct((_NPAD, 128), jnp.float32),
    )(xp, W1p, b1r)


def _segsum_body(h_hbm, src_hbm, dst_hbm, z_hbm, out_hbm,
                 acc, sv0, sv1, sv2, sv3, dv0, dv1, dv2, dv3,
                 rv0, rv1, svt, dvt, rvt, obuf, isem, gsem, ssem):
    c = lax.axis_index("c")
    s = lax.axis_index("s")
    w = c * _NTILES + s
    rbase = s * _RPT
    src_v = [sv0, sv1, sv2, sv3]
    dst_v = [dv0, dv1, dv2, dv3]
    rows_v = [rv0, rv1]

    pltpu.sync_copy(z_hbm, obuf)

    def zk(k, carry):
        pltpu.sync_copy(obuf, acc.at[pl.ds(rbase + k * _RCH, _RCH)])
        return carry
    lax.fori_loop(0, _NRC, zk, 0)
    plsc.subcore_barrier()

    ebase = w * _EPW

    def issue_idx(j, bi):
        base = ebase + j * _CH
        pltpu.async_copy(src_hbm.at[pl.ds(base, _CH)], src_v[bi],
                         isem.at[bi])
        pltpu.async_copy(dst_hbm.at[pl.ds(base, _CH)], dst_v[bi],
                         isem.at[bi])

    def wait_idx(j, bi):
        base = ebase + j * _CH
        pltpu.make_async_copy(src_hbm.at[pl.ds(base, _CH)], src_v[bi],
                              isem.at[bi]).wait()
        pltpu.make_async_copy(dst_hbm.at[pl.ds(base, _CH)], dst_v[bi],
                              isem.at[bi]).wait()

    def issue_gather(bi, br):
        pltpu.async_copy(h_hbm.at[src_v[bi]], rows_v[br], gsem.at[br])

    def wait_gather(bi, br):
        pltpu.make_async_copy(h_hbm.at[src_v[bi]], rows_v[br],
                              gsem.at[br]).wait()

    def issue_scatter(bi, br):
        pltpu.async_copy(rows_v[br], acc.at[dst_v[bi]], ssem.at[br],
                         add=True)

    def wait_scatter(bi, br):
        pltpu.make_async_copy(rows_v[br], acc.at[dst_v[bi]],
                              ssem.at[br]).wait()

    issue_idx(0, 0)
    issue_idx(1, 1)
    issue_idx(2, 2)
    wait_idx(0, 0)
    issue_gather(0, 0)
    wait_gather(0, 0)
    issue_scatter(0, 0)
    issue_idx(3, 3)
    wait_idx(1, 1)
    issue_gather(1, 1)
    wait_gather(1, 1)
    issue_scatter(1, 1)
    wait_scatter(0, 0)
    issue_idx(4, 0)
    wait_idx(2, 2)
    issue_gather(2, 0)
    wait_gather(2, 0)
    issue_scatter(2, 0)
    wait_scatter(1, 1)
    issue_idx(5, 1)
    wait_idx(3, 3)
    issue_gather(3, 1)
    wait_gather(3, 1)
    issue_scatter(3, 1)
    wait_scatter(2, 0)
    issue_idx(6, 2)
    wait_idx(4, 0)
    issue_gather(0, 0)

    def group(jo, carry):
        for b in range(4):
            j = jo * 4 + b
            br = b % 2
            wait_gather(b, br)
            issue_scatter(b, br)
            wait_scatter((b - 1) % 4, br ^ 1)
            issue_idx(j + 3, (b - 1) % 4)
            wait_idx(j + 1, (b + 1) % 4)
            issue_gather((b + 1) % 4, br ^ 1)
        return carry
    lax.fori_loop(1, (_NP - 4) // 4, group, 0)

    wait_gather(0, 0)
    issue_scatter(0, 0)
    wait_scatter(3, 1)
    issue_idx(_NP - 1, 3)
    wait_idx(_NP - 3, 1)
    issue_gather(1, 1)
    wait_gather(1, 1)
    issue_scatter(1, 1)
    wait_scatter(0, 0)
    wait_idx(_NP - 2, 2)
    issue_gather(2, 0)
    wait_gather(2, 0)
    issue_scatter(2, 0)
    wait_scatter(1, 1)
    wait_idx(_NP - 1, 3)
    issue_gather(3, 1)
    wait_gather(3, 1)
    issue_scatter(3, 1)
    wait_scatter(2, 0)
    wait_scatter(3, 1)

    for base in (ebase + _NP * _CH, ebase + (_NP + 1) * _CH):
        pltpu.sync_copy(src_hbm.at[pl.ds(base, _CH)], sv0)
        pltpu.sync_copy(dst_hbm.at[pl.ds(base, _CH)], dv0)
        pltpu.async_copy(h_hbm.at[sv0], rv0, gsem.at[0]).wait()
        pltpu.sync_copy(rv0, acc.at[dv0], add=True)
    tb = ebase + (_NP + 2) * _CH
    pltpu.sync_copy(src_hbm.at[pl.ds(tb, _CT)], svt)
    pltpu.sync_copy(dst_hbm.at[pl.ds(tb, _CT)], dvt)
    pltpu.async_copy(h_hbm.at[svt], rvt, gsem.at[0]).wait()
    pltpu.sync_copy(rvt, acc.at[dvt], add=True)
    plsc.subcore_barrier()

    def ok(k, carry):
        r = rbase + k * _RCH
        pltpu.sync_copy(acc.at[pl.ds(r, _RCH)], obuf)
        pltpu.sync_copy(obuf, out_hbm.at[c, pl.ds(r, _RCH)])
        return carry
    lax.fori_loop(0, _NRC, ok, 0)


def _segsum(h, src, dst, zrows):
    mesh = plsc.VectorSubcoreMesh(core_axis_name="c", subcore_axis_name="s")
    ivec = pltpu.VMEM((_CH,), jnp.int32)
    rbuf = pltpu.VMEM((_CH, 128), jnp.float32)
    f = pl.kernel(
        _segsum_body,
        mesh=mesh,
        out_type=jax.ShapeDtypeStruct((_NCORES, _NPAD, 128), jnp.float32),
        scratch_types=[
            pltpu.VMEM_SHARED((_NPAD, 128), jnp.float32),
            ivec, ivec, ivec, ivec,
            ivec, ivec, ivec, ivec,
            rbuf, rbuf,
            pltpu.VMEM((_CT,), jnp.int32),
            pltpu.VMEM((_CT,), jnp.int32),
            pltpu.VMEM((_CT, 128), jnp.float32),
            pltpu.VMEM((_RCH, 128), jnp.float32),
            pltpu.SemaphoreType.DMA((4,)),
            pltpu.SemaphoreType.DMA((2,)),
            pltpu.SemaphoreType.DMA((2,)),
        ],
    )
    return f(h, src, dst, zrows)


def _tail_body(h_ref, s0_ref, s1_ref, bb_ref, wr_ref, wn_ref, bc_ref,
               w2_ref, b2_ref, wmu_ref, bmu_ref, wlv_ref, blv_ref,
               mu_ref, lv_ref, acc_ref):
    i = pl.program_id(0)
    nsteps = pl.num_programs(0)

    @pl.when(i == 0)
    def _():
        acc_ref[...] = jnp.full((_G, 128), -jnp.inf, jnp.float32)

    h = h_ref[...]
    sagg = s0_ref[0] + s1_ref[0]
    h2 = (jnp.dot(h, wr_ref[...], preferred_element_type=jnp.float32)
          + jnp.dot(sagg, wn_ref[...], preferred_element_type=jnp.float32)
          + bc_ref[...])
    h3 = jnp.maximum(
        jnp.dot(h2, w2_ref[...], preferred_element_type=jnp.float32)
        + b2_ref[...], 0.0)

    bb = bb_ref[...]
    glo = bb[0, 0]
    ghi = bb[bb.shape[0] - 1, 0]

    def gstep(g, carry):
        red = jnp.max(jnp.where(bb == g, h3, -jnp.inf), axis=0,
                      keepdims=True)
        cur = acc_ref[pl.ds(g, 1), :]
        acc_ref[pl.ds(g, 1), :] = jnp.maximum(cur, red)
        return carry
    lax.fori_loop(glo, ghi + 1, gstep, 0)

    @pl.when(i == nsteps - 1)
    def _():
        pooled = acc_ref[...]
        mu_ref[...] = (jnp.dot(pooled, wmu_ref[...],
                               preferred_element_type=jnp.float32)
                       + bmu_ref[...])
        lv_ref[...] = (jnp.dot(pooled, wlv_ref[...],
                               preferred_element_type=jnp.float32)
                       + blv_ref[...])


def _tail(h, partials, bb, Wroot, Wnbr, bcr, W2, b2r, Wmu, bmur, Wlv, blvr):
    nb = 2000
    full = lambda r, c: pl.BlockSpec((r, c), lambda i: (0, 0))
    return pl.pallas_call(
        _tail_body,
        grid=(_N // nb,),
        in_specs=[
            pl.BlockSpec((nb, 128), lambda i: (i, 0)),
            pl.BlockSpec((1, nb, 128), lambda i: (0, i, 0)),
            pl.BlockSpec((1, nb, 128), lambda i: (1, i, 0)),
            pl.BlockSpec((nb, 128), lambda i: (i, 0)),
            full(128, 256),
            full(128, 256),
            full(1, 256),
            full(256, 128),
            full(1, 128),
            full(128, 128),
            full(1, 128),
            full(128, 128),
            full(1, 128),
        ],
        out_specs=[full(_G, 128), full(_G, 128)],
        out_shape=[jax.ShapeDtypeStruct((_G, 128), jnp.float32),
                   jax.ShapeDtypeStruct((_G, 128), jnp.float32)],
        scratch_shapes=[pltpu.VMEM((_G, 128), jnp.float32)],
    )(h, partials, partials, bb, Wroot, Wnbr, bcr, W2, b2r, Wmu, bmur,
      Wlv, blvr)


def kernel(x, edge_index, batch, W1, b1, Wroot, Wnbr, bconv, W2, b2,
           Wmu, bmu, Wlv, blv):
    src = edge_index[0]
    dst = edge_index[1]
    xp = jnp.pad(x, ((0, _NPAD - _N), (0, 5)))
    W1p = jnp.pad(W1, ((0, 5), (0, 0)))
    h = _fc1(xp, W1p, b1.reshape(1, 128))
    zrows = jnp.zeros((_RCH, 128), jnp.float32)
    partials = _segsum(h, src, dst, zrows)
    bb = jnp.broadcast_to(batch[:, None], (_N, 128))
    mu, lv = _tail(h, partials, bb, Wroot, Wnbr,
                   bconv.reshape(1, 256), W2, b2.reshape(1, 128),
                   Wmu, bmu.reshape(1, 128), Wlv, blv.reshape(1, 128))
    return (mu, lv)
